# baseline (speedup 1.0000x reference)
import jax
import jax.numpy as jnp
from jax import lax
from jax.experimental import pallas as pl
from jax.experimental.pallas import tpu as pltpu
from jax.experimental.pallas import tpu_sc as plsc

N_NODES = 10000
NP = 10240
IN_CH = 128
HID = 16
HEADS = 8
C1 = HEADS * HID
C2 = 64
CW = 128
E_RAW = 320000
E_TOT = E_RAW + N_NODES

NC = 2
NS = 16
NW = NC * NS
K1 = 96
K2 = 128


def _chunks(k):
    c = -(-E_TOT // (NW * k))
    return c + c % 2


E_PAD = max(_chunks(k) * NW * k for k in (K1, K2))

BN = 256


def _pre1_body(x_ref, w_ref, a_ref, h_ref, ab_ref):
    h = jnp.dot(x_ref[...], w_ref[...], preferred_element_type=jnp.float32)
    h_ref[...] = h
    ab_ref[...] = jnp.dot(h, a_ref[...], preferred_element_type=jnp.float32)


def _tc_pre1(xp, W1, A1):
    return pl.pallas_call(
        _pre1_body,
        grid=(NP // BN,),
        in_specs=[
            pl.BlockSpec((BN, IN_CH), lambda i: (i, 0)),
            pl.BlockSpec((IN_CH, C1), lambda i: (0, 0)),
            pl.BlockSpec((C1, 16), lambda i: (0, 0)),
        ],
        out_specs=[
            pl.BlockSpec((BN, C1), lambda i: (i, 0)),
            pl.BlockSpec((BN, 16), lambda i: (i, 0)),
        ],
        out_shape=[
            jax.ShapeDtypeStruct((NP, C1), jnp.float32),
            jax.ShapeDtypeStruct((NP, 16), jnp.float32),
        ],
    )(xp, W1, A1)


def _mid_body(acc_ref, sum_ref, exp_ref, b1_ref, w2_ref, a2_ref, h2_ref, ab2_ref):
    acc = acc_ref[0] + acc_ref[1]
    sm = sum_ref[0] + sum_ref[1] + 1e-16
    den = jnp.dot(sm, exp_ref[...], preferred_element_type=jnp.float32)
    h1 = acc / den + b1_ref[...]
    h1 = jnp.where(h1 > 0, h1, jnp.exp(h1) - 1.0)
    h2 = jnp.dot(h1, w2_ref[...], preferred_element_type=jnp.float32)
    h2_ref[...] = jnp.concatenate(
        [h2, jnp.zeros((BN, CW - C2), jnp.float32)], axis=1)
    ab2_ref[...] = jnp.dot(h2, a2_ref[...], preferred_element_type=jnp.float32)


def _tc_mid(acc1, asum1, EXPAND1, b1, W2, A2):
    return pl.pallas_call(
        _mid_body,
        grid=(NP // BN,),
        in_specs=[
            pl.BlockSpec((NC, BN, C1), lambda i: (0, i, 0)),
            pl.BlockSpec((NC, BN, HEADS), lambda i: (0, i, 0)),
            pl.BlockSpec((HEADS, C1), lambda i: (0, 0)),
            pl.BlockSpec((1, C1), lambda i: (0, 0)),
            pl.BlockSpec((C1, C2), lambda i: (0, 0)),
            pl.BlockSpec((C2, 16), lambda i: (0, 0)),
        ],
        out_specs=[
            pl.BlockSpec((BN, CW), lambda i: (i, 0)),
            pl.BlockSpec((BN, 16), lambda i: (i, 0)),
        ],
        out_shape=[
            jax.ShapeDtypeStruct((NP, CW), jnp.float32),
            jax.ShapeDtypeStruct((NP, 16), jnp.float32),
        ],
    )(acc1, asum1, EXPAND1, b1, W2, A2)


def _post_body(acc_ref, sum_ref, b2_ref, o_ref):
    acc = acc_ref[0, :, :C2] + acc_ref[1, :, :C2]
    sm = sum_ref[0] + sum_ref[1] + 1e-16
    o_ref[...] = acc / jnp.broadcast_to(sm, (BN, C2)) + b2_ref[...]


def _tc_post(acc2, asum2, b2):
    return pl.pallas_call(
        _post_body,
        grid=(NP // BN,),
        in_specs=[
            pl.BlockSpec((NC, BN, CW), lambda i: (0, i, 0)),
            pl.BlockSpec((NC, BN, 1), lambda i: (0, i, 0)),
            pl.BlockSpec((1, C2), lambda i: (0, 0)),
        ],
        out_specs=pl.BlockSpec((BN, C2), lambda i: (i, 0)),
        out_shape=jax.ShapeDtypeStruct((NP, C2), jnp.float32),
    )(acc2, asum2, b2)


def _make_sc_edge(AH, H, K):
    mesh = plsc.VectorSubcoreMesh(core_axis_name="c", subcore_axis_name="s")
    RPT = NP // NS
    ZR = 16
    ZB = 1024
    ZW = NP * AH // NS
    per_head = CW // (16 * H) if H > 1 else C2 // 16
    CHUNKS = _chunks(K)

    def body(src_hbm, dst_hbm, h_hbm, as_hbm, ad_hbm,
             acc_out, asum_out,
             srcv0, dstv0, idxS0, idxD0, rows0, S0, D0, ex0,
             srcv1, dstv1, idxS1, idxD1, rows1, S1, D1, ex1,
             zbuf, zbuff, acc_sh, asum_sh,
             sem_r0, sem_s0, sem_d0, sem_w0,
             sem_r1, sem_s1, sem_d1, sem_w1):
        c = lax.axis_index("c")
        s = lax.axis_index("s")
        wid = c * NS + s
        BUF = [
            (srcv0, dstv0, idxS0, idxD0, rows0, S0, D0, ex0,
             sem_r0, sem_s0, sem_d0, sem_w0),
            (srcv1, dstv1, idxS1, idxD1, rows1, S1, D1, ex1,
             sem_r1, sem_s1, sem_d1, sem_w1),
        ]

        @pl.loop(0, ZR)
        def _zb(r):
            for j in range(CW // 16):
                zbuf[r, pl.ds(j * 16, 16)] = jnp.zeros((16,), jnp.float32)

        @pl.loop(0, ZB // 16)
        def _zbf(i):
            zbuff[pl.ds(i * 16, 16)] = jnp.zeros((16,), jnp.float32)

        for b in range(RPT // ZR):
            pltpu.sync_copy(zbuf, acc_sh.at[pl.ds(s * RPT + b * ZR, ZR)])
        for t in range(ZW // ZB):
            pltpu.sync_copy(zbuff, asum_sh.at[pl.ds(s * ZW + t * ZB, ZB)])
        if ZW % ZB:
            pltpu.sync_copy(zbuff.at[pl.ds(0, ZW % ZB)],
                            asum_sh.at[pl.ds(s * ZW + (ZW // ZB) * ZB, ZW % ZB)])
        plsc.subcore_barrier()

        def issue(b, i):
            (srcv, dstv, idxS, idxD, rows, S, D, exbuf,
             sem_r, sem_s, sem_d, sem_w) = BUF[b]
            ebase = (wid * CHUNKS + i) * K
            pltpu.sync_copy(src_hbm.at[pl.ds(ebase, K)], srcv.at[0])
            pltpu.sync_copy(dst_hbm.at[pl.ds(ebase, K)], dstv.at[0])
            pltpu.async_copy(h_hbm.at[srcv.at[0]], rows, sem_r)
            if AH > 1:
                @pl.loop(0, K // 16)
                def _bi(j):
                    sl = pl.ds(j * 16, 16)
                    sv = srcv[0, sl] * AH
                    dv = dstv[0, sl] * AH
                    for g in range(AH):
                        idxS[g, sl] = sv + g
                        idxD[g, sl] = dv + g
                for g in range(AH):
                    pltpu.async_copy(as_hbm.at[idxS.at[g]], S.at[g], sem_s)
                    pltpu.async_copy(ad_hbm.at[idxD.at[g]], D.at[g], sem_d)
            else:
                pltpu.async_copy(as_hbm.at[srcv.at[0]], S.at[0], sem_s)
                pltpu.async_copy(ad_hbm.at[dstv.at[0]], D.at[0], sem_d)

        def drain_scatters(b):
            (srcv, dstv, idxS, idxD, rows, S, D, exbuf,
             sem_r, sem_s, sem_d, sem_w) = BUF[b]
            pltpu.make_async_copy(rows, acc_sh.at[dstv.at[0]], sem_w).wait()
            if AH > 1:
                for g in range(AH):
                    pltpu.make_async_copy(
                        exbuf.at[g], asum_sh.at[idxD.at[g]], sem_w).wait()
            else:
                pltpu.make_async_copy(
                    exbuf.at[0], asum_sh.at[dstv.at[0]], sem_w).wait()

        def proc(b):
            (srcv, dstv, idxS, idxD, rows, S, D, exbuf,
             sem_r, sem_s, sem_d, sem_w) = BUF[b]
            if AH > 1:
                for g in range(AH):
                    pltpu.make_async_copy(
                        as_hbm.at[idxS.at[g]], S.at[g], sem_s).wait()
                    pltpu.make_async_copy(
                        ad_hbm.at[idxD.at[g]], D.at[g], sem_d).wait()
            else:
                pltpu.make_async_copy(
                    as_hbm.at[srcv.at[0]], S.at[0], sem_s).wait()
                pltpu.make_async_copy(
                    ad_hbm.at[dstv.at[0]], D.at[0], sem_d).wait()

            for g in range(AH):
                @pl.loop(0, K // 16)
                def _ex(j):
                    sl = pl.ds(j * 16, 16)
                    al = S[g, sl] + D[g, sl]
                    al = jnp.where(al >= 0, al, 0.2 * al)
                    exbuf[g, sl] = jnp.exp(al)

            pltpu.make_async_copy(h_hbm.at[srcv.at[0]], rows, sem_r).wait()

            @pl.loop(0, K // 16)
            def _app(j):
                base = j * 16
                for g in range(H):
                    ev = exbuf[g, pl.ds(base, 16)]
                    for k in range(16):
                        w16 = ev.at[jnp.full((16,), k, jnp.int32)].get(
                            mode="promise_in_bounds")
                        for kk in range(per_head):
                            sl = pl.ds((g * per_head + kk) * 16, 16)
                            rows[base + k, sl] = rows[base + k, sl] * w16

            pltpu.async_copy(rows, acc_sh.at[dstv.at[0]], sem_w, add=True)
            if AH > 1:
                for g in range(AH):
                    pltpu.async_copy(exbuf.at[g], asum_sh.at[idxD.at[g]],
                                     sem_w, add=True)
            else:
                pltpu.async_copy(exbuf.at[0], asum_sh.at[dstv.at[0]],
                                 sem_w, add=True)

        issue(0, 0)
        issue(1, 1)

        @pl.loop(0, CHUNKS // 2)
        def _round(h):
            i0 = 2 * h
            proc(0)

            @pl.when(i0 + 2 < CHUNKS)
            def _n0():
                drain_scatters(0)
                issue(0, i0 + 2)

            proc(1)

            @pl.when(i0 + 3 < CHUNKS)
            def _n1():
                drain_scatters(1)
                issue(1, i0 + 3)

        drain_scatters(0)
        drain_scatters(1)
        plsc.subcore_barrier()
        pltpu.sync_copy(acc_sh.at[pl.ds(s * RPT, RPT)],
                        acc_out.at[c, pl.ds(s * RPT, RPT)])
        pltpu.sync_copy(asum_sh.at[pl.ds(s * ZW, ZW)],
                        asum_out.at[c, pl.ds(s * ZW, ZW)])

    dbuf = [
        pltpu.VMEM((1, K), jnp.int32),
        pltpu.VMEM((1, K), jnp.int32),
        pltpu.VMEM((AH, K), jnp.int32),
        pltpu.VMEM((AH, K), jnp.int32),
        pltpu.VMEM((K, CW), jnp.float32),
        pltpu.VMEM((AH, K), jnp.float32),
        pltpu.VMEM((AH, K), jnp.float32),
        pltpu.VMEM((AH, K), jnp.float32),
    ]
    return pl.kernel(
        body,
        out_type=[
            jax.ShapeDtypeStruct((NC, NP, CW), jnp.float32),
            jax.ShapeDtypeStruct((NC, NP * AH), jnp.float32),
        ],
        mesh=mesh,
        scratch_types=dbuf + dbuf + [
            pltpu.VMEM((ZR, CW), jnp.float32),
            pltpu.VMEM((ZB,), jnp.float32),
            pltpu.VMEM_SHARED((NP, CW), jnp.float32),
            pltpu.VMEM_SHARED((NP * AH,), jnp.float32),
        ] + [pltpu.SemaphoreType.DMA] * 8,
    )


_sc_edge_l1 = _make_sc_edge(HEADS, HEADS, K1)
_sc_edge_l2 = _make_sc_edge(1, 1, K2)


def kernel(x, edge_index, W1, att_src1, att_dst1, b1, W2, att_src2, att_dst2, b2):
    f32 = jnp.float32
    xp = jnp.zeros((NP, IN_CH), f32).at[:N_NODES].set(x)

    loop_idx = jnp.arange(N_NODES, dtype=jnp.int32)
    pad_e = E_PAD - E_TOT
    src = jnp.concatenate([edge_index[0].astype(jnp.int32), loop_idx,
                           jnp.zeros((pad_e,), jnp.int32)])
    dst = jnp.concatenate([edge_index[1].astype(jnp.int32), loop_idx,
                           jnp.full((pad_e,), N_NODES, jnp.int32)])

    eye8 = jnp.eye(HEADS, dtype=f32)
    A1s = (att_src1[0][:, :, None] * eye8[:, None, :]).reshape(C1, HEADS)
    A1d = (att_dst1[0][:, :, None] * eye8[:, None, :]).reshape(C1, HEADS)
    A1 = jnp.concatenate([A1s, A1d], axis=1)
    EXPAND1 = jnp.repeat(eye8, HID, axis=1)
    A2 = (jnp.zeros((C2, 16), f32)
          .at[:, 0].set(att_src2[0, 0])
          .at[:, 1].set(att_dst2[0, 0]))

    h1, ab1 = _tc_pre1(xp, W1, A1)
    asF1 = ab1[:, :HEADS].reshape(NP * HEADS)
    adF1 = ab1[:, HEADS:].reshape(NP * HEADS)
    acc1, asum1 = _sc_edge_l1(src, dst, h1, asF1, adF1)
    h2p, ab2 = _tc_mid(acc1, asum1.reshape(NC, NP, HEADS), EXPAND1,
                       b1.reshape(1, C1), W2, A2)
    acc2, asum2 = _sc_edge_l2(src, dst, h2p, ab2[:, 0], ab2[:, 1])
    out = _tc_post(acc2, asum2.reshape(NC, NP, 1), b2.reshape(1, C2))
    return out[:N_NODES]


# speedup vs baseline: 1.1505x; 1.1505x over previous
import jax
import jax.numpy as jnp
from jax import lax
from jax.experimental import pallas as pl
from jax.experimental.pallas import tpu as pltpu
from jax.experimental.pallas import tpu_sc as plsc

N_NODES = 10000
NP = 10240
IN_CH = 128
HID = 16
HEADS = 8
C1 = HEADS * HID
C2 = 64
CW = 128
E_RAW = 320000
E_TOT = E_RAW + N_NODES

NC = 2
NS = 16
NW = NC * NS
K1 = 96
K2 = 96


def _chunks(k):
    c = -(-E_TOT // (NW * k))
    return c + c % 2


E_PAD = max(_chunks(k) * NW * k for k in (K1, K2))

BN = 256


def _pre1_body(x_ref, w_ref, a_ref, h_ref, ab_ref):
    h = jnp.dot(x_ref[...], w_ref[...], preferred_element_type=jnp.float32)
    h_ref[...] = h
    ab_ref[...] = jnp.dot(h, a_ref[...], preferred_element_type=jnp.float32)


def _tc_pre1(xp, W1, A1):
    return pl.pallas_call(
        _pre1_body,
        grid=(NP // BN,),
        in_specs=[
            pl.BlockSpec((BN, IN_CH), lambda i: (i, 0)),
            pl.BlockSpec((IN_CH, C1), lambda i: (0, 0)),
            pl.BlockSpec((C1, 16), lambda i: (0, 0)),
        ],
        out_specs=[
            pl.BlockSpec((BN, C1), lambda i: (i, 0)),
            pl.BlockSpec((BN, 16), lambda i: (i, 0)),
        ],
        out_shape=[
            jax.ShapeDtypeStruct((NP, C1), jnp.float32),
            jax.ShapeDtypeStruct((NP, 16), jnp.float32),
        ],
    )(xp, W1, A1)


def _mid_body(acc_ref, sum_ref, exp_ref, b1_ref, w2_ref, a2_ref, h2_ref, ab2_ref):
    acc = acc_ref[0] + acc_ref[1]
    sm = sum_ref[0] + sum_ref[1] + 1e-16
    den = jnp.dot(sm, exp_ref[...], preferred_element_type=jnp.float32)
    h1 = acc / den + b1_ref[...]
    h1 = jnp.where(h1 > 0, h1, jnp.exp(h1) - 1.0)
    h2 = jnp.dot(h1, w2_ref[...], preferred_element_type=jnp.float32)
    h2_ref[...] = jnp.concatenate(
        [h2, jnp.zeros((BN, CW - C2), jnp.float32)], axis=1)
    ab2_ref[...] = jnp.dot(h2, a2_ref[...], preferred_element_type=jnp.float32)


def _tc_mid(acc1, asum1, EXPAND1, b1, W2, A2):
    return pl.pallas_call(
        _mid_body,
        grid=(NP // BN,),
        in_specs=[
            pl.BlockSpec((NC, BN, C1), lambda i: (0, i, 0)),
            pl.BlockSpec((NC, BN, HEADS), lambda i: (0, i, 0)),
            pl.BlockSpec((HEADS, C1), lambda i: (0, 0)),
            pl.BlockSpec((1, C1), lambda i: (0, 0)),
            pl.BlockSpec((C1, C2), lambda i: (0, 0)),
            pl.BlockSpec((C2, 16), lambda i: (0, 0)),
        ],
        out_specs=[
            pl.BlockSpec((BN, CW), lambda i: (i, 0)),
            pl.BlockSpec((BN, 16), lambda i: (i, 0)),
        ],
        out_shape=[
            jax.ShapeDtypeStruct((NP, CW), jnp.float32),
            jax.ShapeDtypeStruct((NP, 16), jnp.float32),
        ],
    )(acc1, asum1, EXPAND1, b1, W2, A2)


def _post_body(acc_ref, sum_ref, b2_ref, o_ref):
    acc = acc_ref[0, :, :C2] + acc_ref[1, :, :C2]
    sm = sum_ref[0] + sum_ref[1] + 1e-16
    o_ref[...] = acc / jnp.broadcast_to(sm, (BN, C2)) + b2_ref[...]


def _tc_post(acc2, asum2, b2):
    return pl.pallas_call(
        _post_body,
        grid=(NP // BN,),
        in_specs=[
            pl.BlockSpec((NC, BN, CW), lambda i: (0, i, 0)),
            pl.BlockSpec((NC, BN, 1), lambda i: (0, i, 0)),
            pl.BlockSpec((1, C2), lambda i: (0, 0)),
        ],
        out_specs=pl.BlockSpec((BN, C2), lambda i: (i, 0)),
        out_shape=jax.ShapeDtypeStruct((NP, C2), jnp.float32),
    )(acc2, asum2, b2)


def _make_sc_edge(AH, H, K):
    mesh = plsc.VectorSubcoreMesh(core_axis_name="c", subcore_axis_name="s")
    RPT = NP // NS
    ZR = 16
    ZB = 1024
    ZW = NP * AH // NS
    per_head = CW // (16 * H) if H > 1 else C2 // 16
    CHUNKS = _chunks(K)

    def body(src_hbm, dst_hbm, h_hbm, as_hbm, ad_hbm,
             acc_out, asum_out,
             srcv0, dstv0, idxS0, idxD0, rows0, S0, D0, ex0,
             srcv1, dstv1, idxS1, idxD1, rows1, S1, D1, ex1,
             zbuf, zbuff, acc_sh, asum_sh,
             sem_r0, sem_s0, sem_d0, sem_w0,
             sem_r1, sem_s1, sem_d1, sem_w1):
        c = lax.axis_index("c")
        s = lax.axis_index("s")
        wid = c * NS + s
        BUF = [
            (srcv0, dstv0, idxS0, idxD0, rows0, S0, D0, ex0,
             sem_r0, sem_s0, sem_d0, sem_w0),
            (srcv1, dstv1, idxS1, idxD1, rows1, S1, D1, ex1,
             sem_r1, sem_s1, sem_d1, sem_w1),
        ]

        @pl.loop(0, ZR)
        def _zb(r):
            for j in range(CW // 16):
                zbuf[r, pl.ds(j * 16, 16)] = jnp.zeros((16,), jnp.float32)

        @pl.loop(0, ZB // 16)
        def _zbf(i):
            zbuff[pl.ds(i * 16, 16)] = jnp.zeros((16,), jnp.float32)

        for b in range(RPT // ZR):
            pltpu.sync_copy(zbuf, acc_sh.at[pl.ds(s * RPT + b * ZR, ZR)])
        for t in range(ZW // ZB):
            pltpu.sync_copy(zbuff, asum_sh.at[pl.ds(s * ZW + t * ZB, ZB)])
        if ZW % ZB:
            pltpu.sync_copy(zbuff.at[pl.ds(0, ZW % ZB)],
                            asum_sh.at[pl.ds(s * ZW + (ZW // ZB) * ZB, ZW % ZB)])
        plsc.subcore_barrier()

        def issue(b, i):
            (srcv, dstv, idxS, idxD, rows, S, D, exbuf,
             sem_r, sem_s, sem_d, sem_w) = BUF[b]
            ebase = (wid * CHUNKS + i) * K
            pltpu.sync_copy(src_hbm.at[pl.ds(ebase, K)], srcv.at[0])
            pltpu.sync_copy(dst_hbm.at[pl.ds(ebase, K)], dstv.at[0])
            pltpu.async_copy(h_hbm.at[srcv.at[0]], rows, sem_r)
            if AH > 1:
                @pl.loop(0, K // 16)
                def _bi(j):
                    sl = pl.ds(j * 16, 16)
                    sv = srcv[0, sl] * AH
                    dv = dstv[0, sl] * AH
                    for g in range(AH):
                        idxS[g, sl] = sv + g
                        idxD[g, sl] = dv + g
                for g in range(AH):
                    pltpu.async_copy(as_hbm.at[idxS.at[g]], S.at[g], sem_s)
                    pltpu.async_copy(ad_hbm.at[idxD.at[g]], D.at[g], sem_d)
            else:
                pltpu.async_copy(as_hbm.at[srcv.at[0]], S.at[0], sem_s)
                pltpu.async_copy(ad_hbm.at[dstv.at[0]], D.at[0], sem_d)

        def drain_scatters(b):
            (srcv, dstv, idxS, idxD, rows, S, D, exbuf,
             sem_r, sem_s, sem_d, sem_w) = BUF[b]
            pltpu.make_async_copy(rows, acc_sh.at[dstv.at[0]], sem_w).wait()
            if AH > 1:
                for g in range(AH):
                    pltpu.make_async_copy(
                        exbuf.at[g], asum_sh.at[idxD.at[g]], sem_w).wait()
            else:
                pltpu.make_async_copy(
                    exbuf.at[0], asum_sh.at[dstv.at[0]], sem_w).wait()

        def proc(b):
            (srcv, dstv, idxS, idxD, rows, S, D, exbuf,
             sem_r, sem_s, sem_d, sem_w) = BUF[b]
            if AH > 1:
                for g in range(AH):
                    pltpu.make_async_copy(
                        as_hbm.at[idxS.at[g]], S.at[g], sem_s).wait()
                    pltpu.make_async_copy(
                        ad_hbm.at[idxD.at[g]], D.at[g], sem_d).wait()
            else:
                pltpu.make_async_copy(
                    as_hbm.at[srcv.at[0]], S.at[0], sem_s).wait()
                pltpu.make_async_copy(
                    ad_hbm.at[dstv.at[0]], D.at[0], sem_d).wait()

            for g in range(AH):
                @pl.loop(0, K // 16)
                def _ex(j):
                    sl = pl.ds(j * 16, 16)
                    al = S[g, sl] + D[g, sl]
                    al = jnp.where(al >= 0, al, 0.2 * al)
                    exbuf[g, sl] = jnp.exp(al)

            pltpu.make_async_copy(h_hbm.at[srcv.at[0]], rows, sem_r).wait()

            @pl.loop(0, K // 16)
            def _app(j):
                base = j * 16
                for g in range(H):
                    ev = exbuf[g, pl.ds(base, 16)]
                    for k in range(16):
                        w16 = ev.at[jnp.full((16,), k, jnp.int32)].get(
                            mode="promise_in_bounds")
                        for kk in range(per_head):
                            sl = pl.ds((g * per_head + kk) * 16, 16)
                            rows[base + k, sl] = rows[base + k, sl] * w16

            pltpu.async_copy(rows, acc_sh.at[dstv.at[0]], sem_w, add=True)
            if AH > 1:
                for g in range(AH):
                    pltpu.async_copy(exbuf.at[g], asum_sh.at[idxD.at[g]],
                                     sem_w, add=True)
            else:
                pltpu.async_copy(exbuf.at[0], asum_sh.at[dstv.at[0]],
                                 sem_w, add=True)

        issue(0, 0)
        issue(1, 1)

        @pl.loop(0, CHUNKS // 2)
        def _round(h):
            i0 = 2 * h
            proc(0)

            @pl.when(i0 + 2 < CHUNKS)
            def _n0():
                drain_scatters(0)
                issue(0, i0 + 2)

            proc(1)

            @pl.when(i0 + 3 < CHUNKS)
            def _n1():
                drain_scatters(1)
                issue(1, i0 + 3)

        drain_scatters(0)
        drain_scatters(1)
        plsc.subcore_barrier()
        pltpu.sync_copy(acc_sh.at[pl.ds(s * RPT, RPT)],
                        acc_out.at[c, pl.ds(s * RPT, RPT)])
        pltpu.sync_copy(asum_sh.at[pl.ds(s * ZW, ZW)],
                        asum_out.at[c, pl.ds(s * ZW, ZW)])

    dbuf = [
        pltpu.VMEM((1, K), jnp.int32),
        pltpu.VMEM((1, K), jnp.int32),
        pltpu.VMEM((AH, K), jnp.int32),
        pltpu.VMEM((AH, K), jnp.int32),
        pltpu.VMEM((K, CW), jnp.float32),
        pltpu.VMEM((AH, K), jnp.float32),
        pltpu.VMEM((AH, K), jnp.float32),
        pltpu.VMEM((AH, K), jnp.float32),
    ]
    return pl.kernel(
        body,
        out_type=[
            jax.ShapeDtypeStruct((NC, NP, CW), jnp.float32),
            jax.ShapeDtypeStruct((NC, NP * AH), jnp.float32),
        ],
        mesh=mesh,
        scratch_types=dbuf + dbuf + [
            pltpu.VMEM((ZR, CW), jnp.float32),
            pltpu.VMEM((ZB,), jnp.float32),
            pltpu.VMEM_SHARED((NP, CW), jnp.float32),
            pltpu.VMEM_SHARED((NP * AH,), jnp.float32),
        ] + [pltpu.SemaphoreType.DMA] * 8,
    )


_sc_edge_l1 = _make_sc_edge(HEADS, HEADS, K1)
_sc_edge_l2 = _make_sc_edge(1, 1, K2)


def kernel(x, edge_index, W1, att_src1, att_dst1, b1, W2, att_src2, att_dst2, b2):
    f32 = jnp.float32
    xp = jnp.zeros((NP, IN_CH), f32).at[:N_NODES].set(x)

    loop_idx = jnp.arange(N_NODES, dtype=jnp.int32)
    pad_e = E_PAD - E_TOT
    src = jnp.concatenate([edge_index[0].astype(jnp.int32), loop_idx,
                           jnp.zeros((pad_e,), jnp.int32)])
    dst = jnp.concatenate([
        edge_index[1].astype(jnp.int32), loop_idx,
        N_NODES + (jnp.arange(pad_e, dtype=jnp.int32) % (NP - N_NODES))])

    eye8 = jnp.eye(HEADS, dtype=f32)
    A1s = (att_src1[0][:, :, None] * eye8[:, None, :]).reshape(C1, HEADS)
    A1d = (att_dst1[0][:, :, None] * eye8[:, None, :]).reshape(C1, HEADS)
    A1 = jnp.concatenate([A1s, A1d], axis=1)
    EXPAND1 = jnp.repeat(eye8, HID, axis=1)
    A2 = (jnp.zeros((C2, 16), f32)
          .at[:, 0].set(att_src2[0, 0])
          .at[:, 1].set(att_dst2[0, 0]))

    h1, ab1 = _tc_pre1(xp, W1, A1)
    asF1 = ab1[:, :HEADS].reshape(NP * HEADS)
    adF1 = ab1[:, HEADS:].reshape(NP * HEADS)
    acc1, asum1 = _sc_edge_l1(src, dst, h1, asF1, adF1)
    h2p, ab2 = _tc_mid(acc1, asum1.reshape(NC, NP, HEADS), EXPAND1,
                       b1.reshape(1, C1), W2, A2)
    acc2, asum2 = _sc_edge_l2(src, dst, h2p, ab2[:, 0], ab2[:, 1])
    out = _tc_post(acc2, asum2.reshape(NC, NP, 1), b2.reshape(1, C2))
    return out[:N_NODES]


# speedup vs baseline: 1.1572x; 1.0058x over previous
import jax
import jax.numpy as jnp
from jax import lax
from jax.experimental import pallas as pl
from jax.experimental.pallas import tpu as pltpu
from jax.experimental.pallas import tpu_sc as plsc

N_NODES = 10000
NP = 10240
IN_CH = 128
HID = 16
HEADS = 8
C1 = HEADS * HID
C2 = 64
CW = 128
E_RAW = 320000
E_TOT = E_RAW + N_NODES

NC = 2
NS = 16
NW = NC * NS
K1 = 96
K2 = 96


def _chunks(k):
    c = -(-E_TOT // (NW * k))
    return c + c % 2


E_PAD = max(_chunks(k) * NW * k for k in (K1, K2))

BN = 256


def _pre1_body(x_ref, w_ref, a_ref, h_ref, ab_ref):
    h = jnp.dot(x_ref[...], w_ref[...], preferred_element_type=jnp.float32)
    h_ref[...] = h
    ab_ref[...] = jnp.dot(h, a_ref[...], preferred_element_type=jnp.float32)


def _tc_pre1(xp, W1, A1):
    return pl.pallas_call(
        _pre1_body,
        grid=(NP // BN,),
        in_specs=[
            pl.BlockSpec((BN, IN_CH), lambda i: (i, 0)),
            pl.BlockSpec((IN_CH, C1), lambda i: (0, 0)),
            pl.BlockSpec((C1, 16), lambda i: (0, 0)),
        ],
        out_specs=[
            pl.BlockSpec((BN, C1), lambda i: (i, 0)),
            pl.BlockSpec((BN, 16), lambda i: (i, 0)),
        ],
        out_shape=[
            jax.ShapeDtypeStruct((NP, C1), jnp.float32),
            jax.ShapeDtypeStruct((NP, 16), jnp.float32),
        ],
    )(xp, W1, A1)


def _mid_body(acc_ref, sum_ref, exp_ref, b1_ref, w2_ref, a2_ref, h2_ref, ab2_ref):
    acc = acc_ref[0] + acc_ref[1]
    sm = sum_ref[0] + sum_ref[1] + 1e-16
    den = jnp.dot(sm, exp_ref[...], preferred_element_type=jnp.float32)
    h1 = acc / den + b1_ref[...]
    h1 = jnp.where(h1 > 0, h1, jnp.exp(h1) - 1.0)
    h2 = jnp.dot(h1, w2_ref[...], preferred_element_type=jnp.float32)
    h2_ref[...] = jnp.concatenate(
        [h2, jnp.zeros((BN, CW - C2), jnp.float32)], axis=1)
    ab2_ref[...] = jnp.dot(h2, a2_ref[...], preferred_element_type=jnp.float32)


def _tc_mid(acc1, asum1, EXPAND1, b1, W2, A2):
    return pl.pallas_call(
        _mid_body,
        grid=(NP // BN,),
        in_specs=[
            pl.BlockSpec((NC, BN, C1), lambda i: (0, i, 0)),
            pl.BlockSpec((NC, BN, HEADS), lambda i: (0, i, 0)),
            pl.BlockSpec((HEADS, C1), lambda i: (0, 0)),
            pl.BlockSpec((1, C1), lambda i: (0, 0)),
            pl.BlockSpec((C1, C2), lambda i: (0, 0)),
            pl.BlockSpec((C2, 16), lambda i: (0, 0)),
        ],
        out_specs=[
            pl.BlockSpec((BN, CW), lambda i: (i, 0)),
            pl.BlockSpec((BN, 16), lambda i: (i, 0)),
        ],
        out_shape=[
            jax.ShapeDtypeStruct((NP, CW), jnp.float32),
            jax.ShapeDtypeStruct((NP, 16), jnp.float32),
        ],
    )(acc1, asum1, EXPAND1, b1, W2, A2)


def _post_body(acc_ref, sum_ref, b2_ref, o_ref):
    acc = acc_ref[0, :, :C2] + acc_ref[1, :, :C2]
    sm = sum_ref[0] + sum_ref[1] + 1e-16
    o_ref[...] = acc / jnp.broadcast_to(sm, (BN, C2)) + b2_ref[...]


def _tc_post(acc2, asum2, b2):
    return pl.pallas_call(
        _post_body,
        grid=(NP // BN,),
        in_specs=[
            pl.BlockSpec((NC, BN, CW), lambda i: (0, i, 0)),
            pl.BlockSpec((NC, BN, 1), lambda i: (0, i, 0)),
            pl.BlockSpec((1, C2), lambda i: (0, 0)),
        ],
        out_specs=pl.BlockSpec((BN, C2), lambda i: (i, 0)),
        out_shape=jax.ShapeDtypeStruct((NP, C2), jnp.float32),
    )(acc2, asum2, b2)


def _make_sc_edge(AH, H, K):
    mesh = plsc.VectorSubcoreMesh(core_axis_name="c", subcore_axis_name="s")
    RPT = NP // NS
    ZR = 16
    ZB = 1024
    ZW = NP * AH // NS
    per_head = CW // (16 * H) if H > 1 else C2 // 16
    CHUNKS = _chunks(K)

    def body(src_hbm, dst_hbm, h_hbm, as_hbm, ad_hbm,
             acc_out, asum_out,
             srcv0, dstv0, idxS0, idxD0, rows0, S0, D0, ex0,
             srcv1, dstv1, idxS1, idxD1, rows1, S1, D1, ex1,
             zbuf, zbuff, acc_sh, asum_sh,
             sem_r0, sem_s0, sem_d0, sem_w0,
             sem_r1, sem_s1, sem_d1, sem_w1):
        c = lax.axis_index("c")
        s = lax.axis_index("s")
        wid = c * NS + s
        BUF = [
            (srcv0, dstv0, idxS0, idxD0, rows0, S0, D0, ex0,
             sem_r0, sem_s0, sem_d0, sem_w0),
            (srcv1, dstv1, idxS1, idxD1, rows1, S1, D1, ex1,
             sem_r1, sem_s1, sem_d1, sem_w1),
        ]

        @pl.loop(0, ZR)
        def _zb(r):
            for j in range(CW // 16):
                zbuf[r, pl.ds(j * 16, 16)] = jnp.zeros((16,), jnp.float32)

        @pl.loop(0, ZB // 16)
        def _zbf(i):
            zbuff[pl.ds(i * 16, 16)] = jnp.zeros((16,), jnp.float32)

        zcps = []
        for b in range(RPT // ZR):
            zcps.append(pltpu.async_copy(
                zbuf, acc_sh.at[pl.ds(s * RPT + b * ZR, ZR)], sem_w0))
        for t in range(ZW // ZB):
            zcps.append(pltpu.async_copy(
                zbuff, asum_sh.at[pl.ds(s * ZW + t * ZB, ZB)], sem_w0))
        if ZW % ZB:
            zcps.append(pltpu.async_copy(
                zbuff.at[pl.ds(0, ZW % ZB)],
                asum_sh.at[pl.ds(s * ZW + (ZW // ZB) * ZB, ZW % ZB)], sem_w0))
        for cp in zcps:
            cp.wait()
        plsc.subcore_barrier()

        def issue(b, i):
            (srcv, dstv, idxS, idxD, rows, S, D, exbuf,
             sem_r, sem_s, sem_d, sem_w) = BUF[b]
            ebase = (wid * CHUNKS + i) * K
            pltpu.sync_copy(src_hbm.at[pl.ds(ebase, K)], srcv.at[0])
            pltpu.sync_copy(dst_hbm.at[pl.ds(ebase, K)], dstv.at[0])
            pltpu.async_copy(h_hbm.at[srcv.at[0]], rows, sem_r)
            if AH > 1:
                @pl.loop(0, K // 16)
                def _bi(j):
                    sl = pl.ds(j * 16, 16)
                    sv = srcv[0, sl] * AH
                    dv = dstv[0, sl] * AH
                    for g in range(AH):
                        idxS[g, sl] = sv + g
                        idxD[g, sl] = dv + g
                for g in range(AH):
                    pltpu.async_copy(as_hbm.at[idxS.at[g]], S.at[g], sem_s)
                    pltpu.async_copy(ad_hbm.at[idxD.at[g]], D.at[g], sem_d)
            else:
                pltpu.async_copy(as_hbm.at[srcv.at[0]], S.at[0], sem_s)
                pltpu.async_copy(ad_hbm.at[dstv.at[0]], D.at[0], sem_d)

        def drain_scatters(b):
            (srcv, dstv, idxS, idxD, rows, S, D, exbuf,
             sem_r, sem_s, sem_d, sem_w) = BUF[b]
            pltpu.make_async_copy(rows, acc_sh.at[dstv.at[0]], sem_w).wait()
            if AH > 1:
                for g in range(AH):
                    pltpu.make_async_copy(
                        exbuf.at[g], asum_sh.at[idxD.at[g]], sem_w).wait()
            else:
                pltpu.make_async_copy(
                    exbuf.at[0], asum_sh.at[dstv.at[0]], sem_w).wait()

        def proc(b):
            (srcv, dstv, idxS, idxD, rows, S, D, exbuf,
             sem_r, sem_s, sem_d, sem_w) = BUF[b]
            if AH > 1:
                for g in range(AH):
                    pltpu.make_async_copy(
                        as_hbm.at[idxS.at[g]], S.at[g], sem_s).wait()
                    pltpu.make_async_copy(
                        ad_hbm.at[idxD.at[g]], D.at[g], sem_d).wait()
            else:
                pltpu.make_async_copy(
                    as_hbm.at[srcv.at[0]], S.at[0], sem_s).wait()
                pltpu.make_async_copy(
                    ad_hbm.at[dstv.at[0]], D.at[0], sem_d).wait()

            for g in range(AH):
                @pl.loop(0, K // 16)
                def _ex(j):
                    sl = pl.ds(j * 16, 16)
                    al = S[g, sl] + D[g, sl]
                    al = jnp.where(al >= 0, al, 0.2 * al)
                    exbuf[g, sl] = jnp.exp(al)

            pltpu.make_async_copy(h_hbm.at[srcv.at[0]], rows, sem_r).wait()

            @pl.loop(0, K // 16, unroll=2)
            def _app(j):
                base = j * 16
                for g in range(H):
                    ev = exbuf[g, pl.ds(base, 16)]
                    for k in range(16):
                        w16 = ev.at[jnp.full((16,), k, jnp.int32)].get(
                            mode="promise_in_bounds")
                        for kk in range(per_head):
                            sl = pl.ds((g * per_head + kk) * 16, 16)
                            rows[base + k, sl] = rows[base + k, sl] * w16

            pltpu.async_copy(rows, acc_sh.at[dstv.at[0]], sem_w, add=True)
            if AH > 1:
                for g in range(AH):
                    pltpu.async_copy(exbuf.at[g], asum_sh.at[idxD.at[g]],
                                     sem_w, add=True)
            else:
                pltpu.async_copy(exbuf.at[0], asum_sh.at[dstv.at[0]],
                                 sem_w, add=True)

        issue(0, 0)
        issue(1, 1)

        @pl.loop(0, CHUNKS // 2)
        def _round(h):
            i0 = 2 * h
            proc(0)

            @pl.when(i0 + 2 < CHUNKS)
            def _n0():
                drain_scatters(0)
                issue(0, i0 + 2)

            proc(1)

            @pl.when(i0 + 3 < CHUNKS)
            def _n1():
                drain_scatters(1)
                issue(1, i0 + 3)

        drain_scatters(0)
        drain_scatters(1)
        plsc.subcore_barrier()
        pltpu.sync_copy(acc_sh.at[pl.ds(s * RPT, RPT)],
                        acc_out.at[c, pl.ds(s * RPT, RPT)])
        pltpu.sync_copy(asum_sh.at[pl.ds(s * ZW, ZW)],
                        asum_out.at[c, pl.ds(s * ZW, ZW)])

    dbuf = [
        pltpu.VMEM((1, K), jnp.int32),
        pltpu.VMEM((1, K), jnp.int32),
        pltpu.VMEM((AH, K), jnp.int32),
        pltpu.VMEM((AH, K), jnp.int32),
        pltpu.VMEM((K, CW), jnp.float32),
        pltpu.VMEM((AH, K), jnp.float32),
        pltpu.VMEM((AH, K), jnp.float32),
        pltpu.VMEM((AH, K), jnp.float32),
    ]
    return pl.kernel(
        body,
        out_type=[
            jax.ShapeDtypeStruct((NC, NP, CW), jnp.float32),
            jax.ShapeDtypeStruct((NC, NP * AH), jnp.float32),
        ],
        mesh=mesh,
        scratch_types=dbuf + dbuf + [
            pltpu.VMEM((ZR, CW), jnp.float32),
            pltpu.VMEM((ZB,), jnp.float32),
            pltpu.VMEM_SHARED((NP, CW), jnp.float32),
            pltpu.VMEM_SHARED((NP * AH,), jnp.float32),
        ] + [pltpu.SemaphoreType.DMA] * 8,
    )


_sc_edge_l1 = _make_sc_edge(HEADS, HEADS, K1)
_sc_edge_l2 = _make_sc_edge(1, 1, K2)


def kernel(x, edge_index, W1, att_src1, att_dst1, b1, W2, att_src2, att_dst2, b2):
    f32 = jnp.float32
    xp = jnp.zeros((NP, IN_CH), f32).at[:N_NODES].set(x)

    loop_idx = jnp.arange(N_NODES, dtype=jnp.int32)
    pad_e = E_PAD - E_TOT
    src = jnp.concatenate([edge_index[0].astype(jnp.int32), loop_idx,
                           jnp.zeros((pad_e,), jnp.int32)])
    dst = jnp.concatenate([
        edge_index[1].astype(jnp.int32), loop_idx,
        N_NODES + (jnp.arange(pad_e, dtype=jnp.int32) % (NP - N_NODES))])

    eye8 = jnp.eye(HEADS, dtype=f32)
    A1s = (att_src1[0][:, :, None] * eye8[:, None, :]).reshape(C1, HEADS)
    A1d = (att_dst1[0][:, :, None] * eye8[:, None, :]).reshape(C1, HEADS)
    A1 = jnp.concatenate([A1s, A1d], axis=1)
    EXPAND1 = jnp.repeat(eye8, HID, axis=1)
    A2 = (jnp.zeros((C2, 16), f32)
          .at[:, 0].set(att_src2[0, 0])
          .at[:, 1].set(att_dst2[0, 0]))

    h1, ab1 = _tc_pre1(xp, W1, A1)
    asF1 = ab1[:, :HEADS].reshape(NP * HEADS)
    adF1 = ab1[:, HEADS:].reshape(NP * HEADS)
    acc1, asum1 = _sc_edge_l1(src, dst, h1, asF1, adF1)
    h2p, ab2 = _tc_mid(acc1, asum1.reshape(NC, NP, HEADS), EXPAND1,
                       b1.reshape(1, C1), W2, A2)
    acc2, asum2 = _sc_edge_l2(src, dst, h2p, ab2[:, 0], ab2[:, 1])
    out = _tc_post(acc2, asum2.reshape(NC, NP, 1), b2.reshape(1, C2))
    return out[:N_NODES]
